# baseline (speedup 1.0000x reference)
import functools

import jax
import jax.numpy as jnp
from jax import lax
from jax.experimental import pallas as pl
from jax.experimental.pallas import tpu as pltpu
from jax.experimental.pallas import tpu_sc as plsc

B = 16384
D = 64
E = 1000000
E_STREAM = 999936
NC = 2
NS = 16
NW = NC * NS
NIDS = 2 * B
CH_E = 256
WCH = 123
WSPAN = WCH * CH_E
PIECE = 4096
SCRATCH_ROWS = NIDS + 16
DUMP = NIDS
BCAP = NIDS + 64
HW = (WCH + 2) * 16
NBUF = 4
NSTG = 4
BPW = B // NW
SUB = 128

_mesh = plsc.VectorSubcoreMesh(core_axis_name="c", subcore_axis_name="s")
_params = pltpu.CompilerParams(needs_layout_passes=False)


@functools.partial(
    pl.kernel,
    mesh=_mesh,
    out_type=jax.ShapeDtypeStruct((SCRATCH_ROWS, 128), jnp.float32),
    compiler_params=_params,
    scratch_types=[
        pltpu.VMEM((PIECE,), jnp.int32),
        pltpu.VMEM((BCAP,), jnp.int32),
        pltpu.VMEM((HW,), jnp.int32),
        pltpu.VMEM((HW,), jnp.int32),
        pltpu.VMEM((HW,), jnp.int32),
        pltpu.VMEM((NBUF, D, CH_E), jnp.float32),
        pltpu.VMEM((NSTG, 16, 128), jnp.float32),
        pltpu.VMEM((NSTG, 16), jnp.int32),
        pltpu.SemaphoreType.DMA,
        pltpu.SemaphoreType.DMA,
    ],
)
def _gather_sc(ids_hbm, ent_t, tail_hbm, scratch_hbm,
               ids_buf, buckets, hist, offs, bases, cbuf, stage, posbuf,
               dsem, ssem):
    wid = lax.axis_index("s") * NC + lax.axis_index("c")
    wstart = wid * WSPAN
    wend = jnp.minimum(wstart + WSPAN, E)
    nch = (jnp.minimum(wend, E_STREAM) - wstart + CH_E - 1) // CH_E
    lane = lax.iota(jnp.int32, 16)
    ones = jnp.ones((16,), jnp.int32)
    zeros = jnp.zeros((16,), jnp.int32)

    def zero_hist(i, carry):
        hist[pl.ds(i * 16, 16)] = zeros
        return carry

    lax.fori_loop(0, HW // 16, zero_hist, 0)

    def count_piece(p, carry):
        pltpu.sync_copy(ids_hbm.at[p], ids_buf)

        def g_body(g, carry):
            v = ids_buf[pl.ds(g * 16, 16)]
            m = (v >= wstart) & (v < wend)
            b = jnp.clip((v - wstart) >> 8, 0, WCH)
            plsc.addupdate_scatter(hist, [b * 16 + lane], ones, mask=m)
            return carry

        return lax.fori_loop(0, PIECE // 16, g_body, carry)

    lax.fori_loop(0, NIDS // PIECE, count_piece, 0)

    def prefix_row(bb, base):
        row = hist[pl.ds(bb * 16, 16)]
        cum = plsc.cumsum(row)
        offs[pl.ds(bb * 16, 16)] = base + cum - row
        bases[pl.ds(bb * 16, 16)] = base
        return base + jnp.sum(row)

    base = lax.fori_loop(0, WCH + 1, prefix_row, zeros)
    bases[pl.ds((WCH + 1) * 16, 16)] = base

    def place_piece(p, carry):
        pltpu.sync_copy(ids_hbm.at[p], ids_buf)

        def g_body(g, carry):
            v = ids_buf[pl.ds(g * 16, 16)]
            m = (v >= wstart) & (v < wend)
            b = jnp.clip((v - wstart) >> 8, 0, WCH)
            addr = b * 16 + lane
            slot = plsc.load_gather(offs, [addr], mask=m)
            packed = ((v - wstart) << 16) | (p * PIECE + g * 16 + lane)
            plsc.store_scatter(buckets, [slot], packed, mask=m)
            plsc.addupdate_scatter(offs, [addr], ones, mask=m)
            return carry

        return lax.fori_loop(0, PIECE // 16, g_body, carry)

    lax.fori_loop(0, NIDS // PIECE, place_piece, 0)

    def issue(c):
        cs = wstart + c * CH_E
        pltpu.async_copy(ent_t.at[:, pl.ds(cs, CH_E)], cbuf.at[c % NBUF], dsem)

    def wait(c):
        cs = wstart + c * CH_E
        pltpu.make_async_copy(
            ent_t.at[:, pl.ds(cs, CH_E)], cbuf.at[c % NBUF], dsem
        ).wait()

    for kk in range(NBUF):
        @pl.when(kk < nch)
        def _prime():
            issue(kk)

    def extract_bucket(c, par, eg):
        start = bases[pl.ds(c * 16, 16)][0]
        end = bases[pl.ds((c + 1) * 16, 16)][0]

        def e_body(g, eg):
            s = eg % NSTG

            @pl.when(eg >= NSTG)
            def _drain():
                pltpu.make_async_copy(
                    stage.at[s], scratch_hbm.at[posbuf.at[s]], ssem
                ).wait()

            pe = buckets[pl.ds(start + g * 16, 16)]
            valid = (start + g * 16 + lane) < end
            col = (pe >> 16) & (CH_E - 1)
            pslot = posbuf.at[s]
            pslot[...] = jnp.where(valid, pe & 0xFFFF, DUMP)
            for d in range(D):
                dv = jnp.full((16,), d, jnp.int32)
                vals = plsc.load_gather(cbuf.at[par], [dv, col])
                plsc.store_scatter(stage.at[s], [lane, dv], vals)
            pltpu.async_copy(stage.at[s], scratch_hbm.at[posbuf.at[s]], ssem)
            return eg + 1

        return lax.fori_loop(0, (end - start + 15) // 16, e_body, eg)

    def chunk_body(c, eg):
        wait(c)
        eg = extract_bucket(c, c % NBUF, eg)

        @pl.when(c + NBUF < nch)
        def _issue_next():
            issue(c + NBUF)

        return eg

    eg = lax.fori_loop(0, nch, chunk_body, 0)

    def tail_fn(eg):
        pltpu.sync_copy(tail_hbm, cbuf.at[0, :, pl.ds(0, 128)])
        return extract_bucket(nch, 0, eg)

    eg = lax.cond(wend >= E, tail_fn, lambda eg: eg, eg)

    for i in range(NSTG):
        @pl.when(eg > i)
        def _final_drain():
            pltpu.make_async_copy(
                stage.at[i], scratch_hbm.at[posbuf.at[i]], ssem
            ).wait()


@functools.partial(
    pl.kernel,
    mesh=_mesh,
    out_type=jax.ShapeDtypeStruct((B,), jnp.float32),
    compiler_params=_params,
    scratch_types=[
        pltpu.VMEM((BPW // SUB, SUB), jnp.int32),
        pltpu.VMEM((SUB, 128), jnp.float32),
        pltpu.VMEM((SUB, 128), jnp.float32),
        pltpu.VMEM((SUB, 128), jnp.float32),
        pltpu.VMEM((BPW,), jnp.float32),
        pltpu.SemaphoreType.DMA,
    ],
)
def _score_sc(rel_ids_hbm, scratch_hbm, rel128_hbm, out_hbm,
              ridx, srow, orow, rrow, outv, sem):
    wid = lax.axis_index("s") * NC + lax.axis_index("c")
    base = wid * BPW
    pltpu.sync_copy(rel_ids_hbm.at[wid], ridx)
    lane = lax.iota(jnp.int32, 16)

    for j in range(BPW // SUB):
        row0 = base + j * SUB
        c1 = pltpu.async_copy(scratch_hbm.at[pl.ds(row0, SUB)], srow, sem)
        c2 = pltpu.async_copy(scratch_hbm.at[pl.ds(B + row0, SUB)], orow, sem)
        c3 = pltpu.async_copy(rel128_hbm.at[ridx.at[j]], rrow, sem)
        c1.wait()
        c2.wait()
        c3.wait()

        def block(rb, carry):
            row_ids = rb * 16 + lane
            acc = jnp.zeros((16,), jnp.float32)
            for d in range(D):
                cj = jnp.full((16,), d, jnp.int32)
                s = plsc.load_gather(srow, [row_ids, cj])
                r = plsc.load_gather(rrow, [row_ids, cj])
                o = plsc.load_gather(orow, [row_ids, cj])
                dd = s + r - o
                acc = acc + dd * dd
            outv[pl.ds(j * SUB + rb * 16, 16)] = acc
            return carry

        lax.fori_loop(0, SUB // 16, block, 0)

    pltpu.sync_copy(outv, out_hbm.at[pl.ds(base, BPW)])


def kernel(subjects, objects, relations, ent_emb, rel_emb):
    ids = jnp.concatenate(
        [subjects.astype(jnp.int32), objects.astype(jnp.int32)]
    ).reshape(NIDS // PIECE, PIECE)
    rel_ids = relations.astype(jnp.int32).reshape(NW, BPW // SUB, SUB)
    rel128 = jnp.pad(rel_emb, ((0, 0), (0, 128 - D)))
    tail128 = jnp.pad(ent_emb[E_STREAM:].T, ((0, 0), (0, 128 - (E - E_STREAM))))
    scratch = _gather_sc(ids, ent_emb.T, tail128)
    out = _score_sc(rel_ids, scratch, rel128)
    return out.reshape(-1, 1)


# speedup vs baseline: 2.4697x; 2.4697x over previous
import functools

import jax
import jax.numpy as jnp
from jax import lax
from jax.experimental import pallas as pl
from jax.experimental.pallas import tpu as pltpu
from jax.experimental.pallas import tpu_sc as plsc

B = 16384
D = 64
NC = 2
NS = 16
NW = NC * NS
BPW = B // NW
CH = 128
NCH = BPW // CH

_mesh = plsc.VectorSubcoreMesh(core_axis_name="c", subcore_axis_name="s")


@functools.partial(
    pl.kernel,
    mesh=_mesh,
    out_type=jax.ShapeDtypeStruct((B,), jnp.float32),
    compiler_params=pltpu.CompilerParams(
        needs_layout_passes=False, use_tc_tiling_on_sc=False
    ),
    scratch_types=[
        pltpu.VMEM((NCH, CH), jnp.int32),
        pltpu.VMEM((NCH, CH), jnp.int32),
        pltpu.VMEM((NCH, CH), jnp.int32),
        pltpu.VMEM((BPW, D), jnp.float32),
        pltpu.VMEM((BPW, D), jnp.float32),
        pltpu.VMEM((BPW, D), jnp.float32),
        pltpu.VMEM((BPW,), jnp.float32),
        pltpu.SemaphoreType.DMA,
    ],
)
def _transe_sc(sub_hbm, obj_hbm, rel_hbm, ent_hbm, relemb_hbm, out_hbm,
               sidx, oidx, ridx, srow, orow, rrow, outv, sem):
    wid = lax.axis_index("s") * NC + lax.axis_index("c")

    pltpu.sync_copy(sub_hbm.at[wid], sidx)
    pltpu.sync_copy(obj_hbm.at[wid], oidx)
    pltpu.sync_copy(rel_hbm.at[wid], ridx)

    copies = []
    for j in range(NCH):
        dst = pl.ds(j * CH, CH)
        copies.append(pltpu.async_copy(ent_hbm.at[sidx.at[j]], srow.at[dst], sem))
        copies.append(pltpu.async_copy(ent_hbm.at[oidx.at[j]], orow.at[dst], sem))
        copies.append(pltpu.async_copy(relemb_hbm.at[ridx.at[j]], rrow.at[dst], sem))
    for c in copies:
        c.wait()

    lane = lax.iota(jnp.int32, 16)

    def block(rb, carry):
        row_ids = rb * 16 + lane
        acc = jnp.zeros((16,), jnp.float32)
        for j in range(D):
            cj = jnp.full((16,), j, jnp.int32)
            s = plsc.load_gather(srow, [row_ids, cj])
            r = plsc.load_gather(rrow, [row_ids, cj])
            o = plsc.load_gather(orow, [row_ids, cj])
            d = s + r - o
            acc = acc + d * d
        outv[pl.ds(rb * 16, 16)] = acc
        return carry

    lax.fori_loop(0, BPW // 16, block, 0)
    pltpu.sync_copy(outv, out_hbm.at[pl.ds(wid * BPW, BPW)])


def kernel(subjects, objects, relations, ent_emb, rel_emb):
    sub = subjects.astype(jnp.int32).reshape(NW, NCH, CH)
    obj = objects.astype(jnp.int32).reshape(NW, NCH, CH)
    rel = relations.astype(jnp.int32).reshape(NW, NCH, CH)
    out = _transe_sc(sub, obj, rel, ent_emb, rel_emb)
    return out.reshape(-1, 1)


# speedup vs baseline: 2.7342x; 1.1071x over previous
import functools

import jax
import jax.numpy as jnp
from jax import lax
from jax.experimental import pallas as pl
from jax.experimental.pallas import tpu as pltpu
from jax.experimental.pallas import tpu_sc as plsc

B = 16384
D = 64
NC = 2
NS = 16
NW = NC * NS
BPW = B // NW
SUB = 128
NCH = BPW // SUB

_mesh = plsc.VectorSubcoreMesh(core_axis_name="c", subcore_axis_name="s")


@functools.partial(
    pl.kernel,
    mesh=_mesh,
    out_type=jax.ShapeDtypeStruct((B,), jnp.float32),
    compiler_params=pltpu.CompilerParams(needs_layout_passes=False),
    scratch_types=[
        pltpu.VMEM((NCH, SUB), jnp.int32),
        pltpu.VMEM((NCH, SUB), jnp.int32),
        pltpu.VMEM((NCH, SUB), jnp.int32),
        pltpu.VMEM((2, SUB, 128), jnp.float32),
        pltpu.VMEM((2, SUB, 128), jnp.float32),
        pltpu.VMEM((2, SUB, 128), jnp.float32),
        pltpu.VMEM((BPW,), jnp.float32),
        pltpu.SemaphoreType.DMA,
    ],
)
def _transe_sc(sub_hbm, obj_hbm, rel_hbm, ent_hbm, relemb_hbm, out_hbm,
               sidx, oidx, ridx, srow, orow, rrow, outv, sem):
    wid = lax.axis_index("s") * NC + lax.axis_index("c")

    pltpu.sync_copy(sub_hbm.at[wid], sidx)
    pltpu.sync_copy(obj_hbm.at[wid], oidx)
    pltpu.sync_copy(rel_hbm.at[wid], ridx)

    lane = lax.iota(jnp.int32, 16)

    def issue(j):
        pltpu.async_copy(ent_hbm.at[sidx.at[j]], srow.at[j % 2], sem)
        pltpu.async_copy(ent_hbm.at[oidx.at[j]], orow.at[j % 2], sem)
        pltpu.async_copy(relemb_hbm.at[ridx.at[j]], rrow.at[j % 2], sem)

    def wait(j):
        pltpu.make_async_copy(ent_hbm.at[sidx.at[j]], srow.at[j % 2], sem).wait()
        pltpu.make_async_copy(ent_hbm.at[oidx.at[j]], orow.at[j % 2], sem).wait()
        pltpu.make_async_copy(relemb_hbm.at[ridx.at[j]], rrow.at[j % 2], sem).wait()

    issue(0)
    for j in range(NCH):
        wait(j)
        if j + 1 < NCH:
            issue(j + 1)

        def block(rb, carry):
            row_ids = rb * 16 + lane
            acc = jnp.zeros((16,), jnp.float32)
            for d in range(D):
                cj = jnp.full((16,), d, jnp.int32)
                s = plsc.load_gather(srow.at[j % 2], [row_ids, cj])
                r = plsc.load_gather(rrow.at[j % 2], [row_ids, cj])
                o = plsc.load_gather(orow.at[j % 2], [row_ids, cj])
                dd = s + r - o
                acc = acc + dd * dd
            outv[pl.ds(j * SUB + rb * 16, 16)] = acc
            return carry

        lax.fori_loop(0, SUB // 16, block, 0)

    pltpu.sync_copy(outv, out_hbm.at[pl.ds(wid * BPW, BPW)])


def kernel(subjects, objects, relations, ent_emb, rel_emb):
    sub = subjects.astype(jnp.int32).reshape(NW, NCH, SUB)
    obj = objects.astype(jnp.int32).reshape(NW, NCH, SUB)
    rel = relations.astype(jnp.int32).reshape(NW, NCH, SUB)
    ent128 = jnp.pad(ent_emb, ((0, 0), (0, 128 - D)))
    rel128 = jnp.pad(rel_emb, ((0, 0), (0, 128 - D)))
    out = _transe_sc(sub, obj, rel, ent128, rel128)
    return out.reshape(-1, 1)


# speedup vs baseline: 4.0072x; 1.4656x over previous
import functools

import jax
import jax.numpy as jnp
from jax import lax
from jax.experimental import pallas as pl
from jax.experimental.pallas import tpu as pltpu
from jax.experimental.pallas import tpu_sc as plsc

B = 16384
D = 64
NC = 2
NS = 16
NW = NC * NS
BPW = B // NW
SUB = 128
NCH = BPW // SUB

_mesh = plsc.VectorSubcoreMesh(core_axis_name="c", subcore_axis_name="s")


@functools.partial(
    pl.kernel,
    mesh=_mesh,
    out_type=jax.ShapeDtypeStruct((B,), jnp.float32),
    compiler_params=pltpu.CompilerParams(needs_layout_passes=False),
    scratch_types=[
        pltpu.VMEM((NCH, SUB), jnp.int32),
        pltpu.VMEM((NCH, SUB), jnp.int32),
        pltpu.VMEM((NCH, SUB), jnp.int32),
        pltpu.VMEM((2, SUB, 128), jnp.float32),
        pltpu.VMEM((2, SUB, 128), jnp.float32),
        pltpu.VMEM((2, SUB, 128), jnp.float32),
        pltpu.VMEM((BPW,), jnp.float32),
        pltpu.SemaphoreType.DMA,
    ],
)
def _transe_sc(sub_hbm, obj_hbm, rel_hbm, ent_hbm, relemb_hbm, out_hbm,
               sidx, oidx, ridx, srow, orow, rrow, outv, sem):
    wid = lax.axis_index("s") * NC + lax.axis_index("c")

    pltpu.sync_copy(sub_hbm.at[wid], sidx)
    pltpu.sync_copy(obj_hbm.at[wid], oidx)
    pltpu.sync_copy(rel_hbm.at[wid], ridx)

    lane = lax.iota(jnp.int32, 16)

    def issue(j):
        pltpu.async_copy(ent_hbm.at[sidx.at[j]], srow.at[j % 2], sem)
        pltpu.async_copy(ent_hbm.at[oidx.at[j]], orow.at[j % 2], sem)
        pltpu.async_copy(relemb_hbm.at[ridx.at[j]], rrow.at[j % 2], sem)

    def wait(j):
        pltpu.make_async_copy(ent_hbm.at[sidx.at[j]], srow.at[j % 2], sem).wait()
        pltpu.make_async_copy(ent_hbm.at[oidx.at[j]], orow.at[j % 2], sem).wait()
        pltpu.make_async_copy(relemb_hbm.at[ridx.at[j]], rrow.at[j % 2], sem).wait()

    issue(0)
    for j in range(NCH):
        wait(j)
        if j + 1 < NCH:
            issue(j + 1)

        def block(rb, carry):
            row_ids = rb * 16 + lane
            acc = jnp.zeros((16,), jnp.float32)
            for d in range(D):
                cj = jnp.full((16,), d, jnp.int32)
                s = plsc.load_gather(srow.at[j % 2], [row_ids, cj])
                r = plsc.load_gather(rrow.at[j % 2], [row_ids, cj])
                o = plsc.load_gather(orow.at[j % 2], [row_ids, cj])
                dd = s + r - o
                acc = acc + dd * dd
            outv[pl.ds(j * SUB + rb * 16, 16)] = acc
            return carry

        lax.fori_loop(0, SUB // 16, block, 0)

    pltpu.sync_copy(outv, out_hbm.at[pl.ds(wid * BPW, BPW)])


_TBLK = 4096


def _pad_t_tc(ent_t_ref, out_ref):
    out_ref[:, :D] = ent_t_ref[...].T
    out_ref[:, D:] = jnp.zeros((_TBLK, 128 - D), jnp.float32)


def kernel(subjects, objects, relations, ent_emb, rel_emb):
    sub = subjects.astype(jnp.int32).reshape(NW, NCH, SUB)
    obj = objects.astype(jnp.int32).reshape(NW, NCH, SUB)
    rel = relations.astype(jnp.int32).reshape(NW, NCH, SUB)
    ent128 = pl.pallas_call(
        _pad_t_tc,
        grid=((ent_emb.shape[0] + _TBLK - 1) // _TBLK,),
        in_specs=[pl.BlockSpec((D, _TBLK), lambda i: (0, i))],
        out_specs=pl.BlockSpec((_TBLK, 128), lambda i: (i, 0)),
        out_shape=jax.ShapeDtypeStruct((ent_emb.shape[0], 128), jnp.float32),
    )(ent_emb.T)
    rel128 = jnp.pad(rel_emb, ((0, 0), (0, 128 - D)))
    out = _transe_sc(sub, obj, rel, ent128, rel128)
    return out.reshape(-1, 1)


# speedup vs baseline: 4.8778x; 1.2173x over previous
import functools

import jax
import jax.numpy as jnp
from jax import lax
from jax.experimental import pallas as pl
from jax.experimental.pallas import tpu as pltpu
from jax.experimental.pallas import tpu_sc as plsc

B = 16384
D = 64
NC = 2
NS = 16
NW = NC * NS
BPW = B // NW
SUB = 128
NCH = BPW // SUB

_mesh = plsc.VectorSubcoreMesh(core_axis_name="c", subcore_axis_name="s")


@functools.partial(
    pl.kernel,
    mesh=_mesh,
    out_type=jax.ShapeDtypeStruct((B,), jnp.float32),
    compiler_params=pltpu.CompilerParams(needs_layout_passes=False),
    scratch_types=[
        pltpu.VMEM((NCH, SUB), jnp.int32),
        pltpu.VMEM((NCH, SUB), jnp.int32),
        pltpu.VMEM((NCH, SUB), jnp.int32),
        pltpu.VMEM((2, SUB, 128), jnp.float32),
        pltpu.VMEM((2, SUB, 128), jnp.float32),
        pltpu.VMEM((2, SUB, 128), jnp.float32),
        pltpu.VMEM((BPW,), jnp.float32),
        pltpu.SemaphoreType.DMA,
    ],
)
def _transe_sc(sub_hbm, obj_hbm, rel_hbm, ent_hbm, relemb_hbm, out_hbm,
               sidx, oidx, ridx, srow, orow, rrow, outv, sem):
    wid = lax.axis_index("s") * NC + lax.axis_index("c")

    pltpu.sync_copy(sub_hbm.at[wid], sidx)
    pltpu.sync_copy(obj_hbm.at[wid], oidx)
    pltpu.sync_copy(rel_hbm.at[wid], ridx)

    lane = lax.iota(jnp.int32, 16)

    def issue(j):
        pltpu.async_copy(ent_hbm.at[sidx.at[j]], srow.at[j % 2], sem)
        pltpu.async_copy(ent_hbm.at[oidx.at[j]], orow.at[j % 2], sem)
        pltpu.async_copy(relemb_hbm.at[ridx.at[j]], rrow.at[j % 2], sem)

    def wait(j):
        pltpu.make_async_copy(ent_hbm.at[sidx.at[j]], srow.at[j % 2], sem).wait()
        pltpu.make_async_copy(ent_hbm.at[oidx.at[j]], orow.at[j % 2], sem).wait()
        pltpu.make_async_copy(relemb_hbm.at[ridx.at[j]], rrow.at[j % 2], sem).wait()

    issue(0)
    for j in range(NCH):
        wait(j)
        if j + 1 < NCH:
            issue(j + 1)

        def block(rb, carry):
            row_ids = rb * 16 + lane
            acc = jnp.zeros((16,), jnp.float32)
            for d in range(D):
                cj = jnp.full((16,), d, jnp.int32)
                s = plsc.load_gather(srow.at[j % 2], [row_ids, cj])
                r = plsc.load_gather(rrow.at[j % 2], [row_ids, cj])
                o = plsc.load_gather(orow.at[j % 2], [row_ids, cj])
                dd = s + r - o
                acc = acc + dd * dd
            outv[pl.ds(j * SUB + rb * 16, 16)] = acc
            return carry

        lax.fori_loop(0, SUB // 16, block, 0)

    pltpu.sync_copy(outv, out_hbm.at[pl.ds(wid * BPW, BPW)])


_TBLK = 8192


def _pad_t_tc(ent_t_ref, out_ref):
    out_ref[:, :D] = ent_t_ref[...].T
    out_ref[:, D:] = jnp.zeros((_TBLK, 128 - D), jnp.float32)


def kernel(subjects, objects, relations, ent_emb, rel_emb):
    sub = subjects.astype(jnp.int32).reshape(NW, NCH, SUB)
    obj = objects.astype(jnp.int32).reshape(NW, NCH, SUB)
    rel = relations.astype(jnp.int32).reshape(NW, NCH, SUB)
    ent128 = pl.pallas_call(
        _pad_t_tc,
        grid=((ent_emb.shape[0] + _TBLK - 1) // _TBLK,),
        in_specs=[pl.BlockSpec((D, _TBLK), lambda i: (0, i))],
        out_specs=pl.BlockSpec((_TBLK, 128), lambda i: (i, 0)),
        out_shape=jax.ShapeDtypeStruct((ent_emb.shape[0], 128), jnp.float32),
    )(ent_emb.T)
    rel128 = jnp.pad(rel_emb, ((0, 0), (0, 128 - D)))
    out = _transe_sc(sub, obj, rel, ent128, rel128)
    return out.reshape(-1, 1)


# speedup vs baseline: 5.1666x; 1.0592x over previous
import functools

import jax
import jax.numpy as jnp
from jax import lax
from jax.experimental import pallas as pl
from jax.experimental.pallas import tpu as pltpu
from jax.experimental.pallas import tpu_sc as plsc

B = 16384
D = 64
NC = 2
NS = 16
NW = NC * NS
BPW = B // NW
SUB = 128
NCH = BPW // SUB

_mesh = plsc.VectorSubcoreMesh(core_axis_name="c", subcore_axis_name="s")


@functools.partial(
    pl.kernel,
    mesh=_mesh,
    out_type=jax.ShapeDtypeStruct((B,), jnp.float32),
    compiler_params=pltpu.CompilerParams(needs_layout_passes=False),
    scratch_types=[
        pltpu.VMEM((NCH, SUB), jnp.int32),
        pltpu.VMEM((NCH, SUB), jnp.int32),
        pltpu.VMEM((NCH, SUB), jnp.int32),
        pltpu.VMEM((2, SUB, 128), jnp.float32),
        pltpu.VMEM((2, SUB, 128), jnp.float32),
        pltpu.VMEM((2, SUB, 128), jnp.float32),
        pltpu.VMEM((BPW,), jnp.float32),
        pltpu.SemaphoreType.DMA,
    ],
)
def _transe_sc(sub_hbm, obj_hbm, rel_hbm, ent_hbm, relemb_hbm, out_hbm,
               sidx, oidx, ridx, srow, orow, rrow, outv, sem):
    wid = lax.axis_index("s") * NC + lax.axis_index("c")

    pltpu.sync_copy(sub_hbm.at[wid], sidx)
    pltpu.sync_copy(obj_hbm.at[wid], oidx)
    pltpu.sync_copy(rel_hbm.at[wid], ridx)

    lane = lax.iota(jnp.int32, 16)

    def issue(j):
        pltpu.async_copy(ent_hbm.at[sidx.at[j]], srow.at[j % 2], sem)
        pltpu.async_copy(ent_hbm.at[oidx.at[j]], orow.at[j % 2], sem)
        pltpu.async_copy(relemb_hbm.at[ridx.at[j]], rrow.at[j % 2], sem)

    def wait(j):
        pltpu.make_async_copy(ent_hbm.at[sidx.at[j]], srow.at[j % 2], sem).wait()
        pltpu.make_async_copy(ent_hbm.at[oidx.at[j]], orow.at[j % 2], sem).wait()
        pltpu.make_async_copy(relemb_hbm.at[ridx.at[j]], rrow.at[j % 2], sem).wait()

    issue(0)
    for j in range(NCH):
        wait(j)
        if j + 1 < NCH:
            issue(j + 1)

        def block(rb, carry):
            row_ids = rb * 16 + lane
            acc = jnp.zeros((16,), jnp.float32)
            for d in range(D):
                cj = jnp.full((16,), d, jnp.int32)
                s = plsc.load_gather(srow.at[j % 2], [row_ids, cj])
                r = plsc.load_gather(rrow.at[j % 2], [row_ids, cj])
                o = plsc.load_gather(orow.at[j % 2], [row_ids, cj])
                dd = s + r - o
                acc = acc + dd * dd
            outv[pl.ds(j * SUB + rb * 16, 16)] = acc
            return carry

        lax.fori_loop(0, SUB // 16, block, 0)

    pltpu.sync_copy(outv, out_hbm.at[pl.ds(wid * BPW, BPW)])


_TBLK = 16384


def _pad_t_tc(ent_t_ref, out_ref):
    out_ref[:, :D] = ent_t_ref[...].T
    out_ref[:, D:] = jnp.zeros((_TBLK, 128 - D), jnp.float32)


def kernel(subjects, objects, relations, ent_emb, rel_emb):
    sub = subjects.astype(jnp.int32).reshape(NW, NCH, SUB)
    obj = objects.astype(jnp.int32).reshape(NW, NCH, SUB)
    rel = relations.astype(jnp.int32).reshape(NW, NCH, SUB)
    ent128 = pl.pallas_call(
        _pad_t_tc,
        grid=((ent_emb.shape[0] + _TBLK - 1) // _TBLK,),
        in_specs=[pl.BlockSpec((D, _TBLK), lambda i: (0, i))],
        out_specs=pl.BlockSpec((_TBLK, 128), lambda i: (i, 0)),
        out_shape=jax.ShapeDtypeStruct((ent_emb.shape[0], 128), jnp.float32),
    )(ent_emb.T)
    rel128 = jnp.pad(rel_emb, ((0, 0), (0, 128 - D)))
    out = _transe_sc(sub, obj, rel, ent128, rel128)
    return out.reshape(-1, 1)


# speedup vs baseline: 5.2673x; 1.0195x over previous
import functools

import jax
import jax.numpy as jnp
from jax import lax
from jax.experimental import pallas as pl
from jax.experimental.pallas import tpu as pltpu
from jax.experimental.pallas import tpu_sc as plsc

B = 16384
D = 64
NC = 2
NS = 16
NW = NC * NS
BPW = B // NW
SUB = 128
NCH = BPW // SUB

_mesh = plsc.VectorSubcoreMesh(core_axis_name="c", subcore_axis_name="s")


@functools.partial(
    pl.kernel,
    mesh=_mesh,
    out_type=jax.ShapeDtypeStruct((B,), jnp.float32),
    compiler_params=pltpu.CompilerParams(needs_layout_passes=False),
    scratch_types=[
        pltpu.VMEM((NCH, SUB), jnp.int32),
        pltpu.VMEM((NCH, SUB), jnp.int32),
        pltpu.VMEM((NCH, SUB), jnp.int32),
        pltpu.VMEM((2, SUB, 128), jnp.float32),
        pltpu.VMEM((2, SUB, 128), jnp.float32),
        pltpu.VMEM((2, SUB, 128), jnp.float32),
        pltpu.VMEM((BPW,), jnp.float32),
        pltpu.SemaphoreType.DMA,
    ],
)
def _transe_sc(sub_hbm, obj_hbm, rel_hbm, ent_hbm, relemb_hbm, out_hbm,
               sidx, oidx, ridx, srow, orow, rrow, outv, sem):
    wid = lax.axis_index("s") * NC + lax.axis_index("c")

    pltpu.sync_copy(sub_hbm.at[wid], sidx)
    pltpu.sync_copy(obj_hbm.at[wid], oidx)
    pltpu.sync_copy(rel_hbm.at[wid], ridx)

    lane = lax.iota(jnp.int32, 16)

    def issue(j):
        pltpu.async_copy(ent_hbm.at[sidx.at[j]], srow.at[j % 2], sem)
        pltpu.async_copy(ent_hbm.at[oidx.at[j]], orow.at[j % 2], sem)
        pltpu.async_copy(relemb_hbm.at[ridx.at[j]], rrow.at[j % 2], sem)

    def wait(j):
        pltpu.make_async_copy(ent_hbm.at[sidx.at[j]], srow.at[j % 2], sem).wait()
        pltpu.make_async_copy(ent_hbm.at[oidx.at[j]], orow.at[j % 2], sem).wait()
        pltpu.make_async_copy(relemb_hbm.at[ridx.at[j]], rrow.at[j % 2], sem).wait()

    issue(0)
    for j in range(NCH):
        wait(j)
        if j + 1 < NCH:
            issue(j + 1)

        def block(rb, carry):
            row_ids = rb * 16 + lane
            acc = jnp.zeros((16,), jnp.float32)
            for d in range(D):
                cj = jnp.full((16,), d, jnp.int32)
                s = plsc.load_gather(srow.at[j % 2], [row_ids, cj])
                r = plsc.load_gather(rrow.at[j % 2], [row_ids, cj])
                o = plsc.load_gather(orow.at[j % 2], [row_ids, cj])
                dd = s + r - o
                acc = acc + dd * dd
            outv[pl.ds(j * SUB + rb * 16, 16)] = acc
            return carry

        lax.fori_loop(0, SUB // 16, block, 0)

    pltpu.sync_copy(outv, out_hbm.at[pl.ds(wid * BPW, BPW)])


_TBLK = 32768


def _pad_t_tc(ent_t_ref, out_ref):
    out_ref[:, :D] = ent_t_ref[...].T
    out_ref[:, D:] = jnp.zeros((_TBLK, 128 - D), jnp.float32)


def kernel(subjects, objects, relations, ent_emb, rel_emb):
    sub = subjects.astype(jnp.int32).reshape(NW, NCH, SUB)
    obj = objects.astype(jnp.int32).reshape(NW, NCH, SUB)
    rel = relations.astype(jnp.int32).reshape(NW, NCH, SUB)
    ent128 = pl.pallas_call(
        _pad_t_tc,
        grid=((ent_emb.shape[0] + _TBLK - 1) // _TBLK,),
        in_specs=[pl.BlockSpec((D, _TBLK), lambda i: (0, i))],
        out_specs=pl.BlockSpec((_TBLK, 128), lambda i: (i, 0)),
        out_shape=jax.ShapeDtypeStruct((ent_emb.shape[0], 128), jnp.float32),
    )(ent_emb.T)
    rel128 = jnp.pad(rel_emb, ((0, 0), (0, 128 - D)))
    out = _transe_sc(sub, obj, rel, ent128, rel128)
    return out.reshape(-1, 1)
